# baseline (speedup 1.0000x reference)
import functools

import jax
import jax.numpy as jnp
from jax import lax
from jax.experimental import pallas as pl
from jax.experimental.pallas import tpu as pltpu
from jax.experimental.pallas import tpu_sc as plsc

_N = 10000
_E = 320000
_B = 2
_T = 64
_D = 128
_H = 64

_NPT = 10240
_NPA = 10112
_TRASH = _N
_NC = 2
_NS = 16
_NW = _NC * _NS
_EPW = _E // _NW
_GE = 2000
_NOUT = _EPW // _GE
_CH = 48
_RPT = _NPA // _NS
_PACK = 15
_PMASK = (1 << _PACK) - 1
_CLEN = _EPW + 2 * _CH + 16


def _node_scores_body(ei_ref, w1_ref, w2_ref, us8_ref, out_ref):
    z1 = jnp.maximum(ei_ref[0] @ w1_ref[...], 0.0)
    z2 = z1 @ w2_ref[...]
    zb = z2.astype(jnp.bfloat16)
    p = lax.dot_general(us8_ref[...], zb, (((1,), (1,)), ((), ())),
                        preferred_element_type=jnp.float32)
    out_ref[0, pl.ds(0, 1)] = p[0:1] - p[1:2]
    out_ref[0, pl.ds(1, 1)] = p[2:3] - p[3:4]


def _node_scores(ei_p, W1, W2, us8):
    blk = 1280
    grid = (_B, _NPT // blk)
    return pl.pallas_call(
        _node_scores_body,
        grid=grid,
        in_specs=[
            pl.BlockSpec((1, blk, _T), lambda b, i: (b, i, 0)),
            pl.BlockSpec((_H, _H), lambda b, i: (0, 0)),
            pl.BlockSpec((_H, _H), lambda b, i: (0, 0)),
            pl.BlockSpec((8, _H), lambda b, i: (0, 0)),
        ],
        out_specs=pl.BlockSpec((1, 8, blk), lambda b, i: (b, 0, i)),
        out_shape=jax.ShapeDtypeStruct((_B, 8, _NPT), jnp.float32),
    )(ei_p, W1, W2, us8)


def _sc_body(boff, pk_h, gd_h, ds_h, dt_h, x_h,
             m_h, a_h,
             pk_g, gd_g, mask_g, ds_v, dt_v, comp_v, srcs0, srcs1,
             dste0, dste1, rows0, rows1, agg_sh, sem0, sem1, scat0, scat1):
    c = lax.axis_index("c")
    s = lax.axis_index("s")
    wid = c * _NS + s
    ebase = wid * _EPW
    rbase = s * _RPT

    zv = jnp.zeros((16,), jnp.float32)

    if True:
        pltpu.sync_copy(ds_h.at[pl.ds(0, _N)], ds_v)
        pltpu.sync_copy(dt_h.at[pl.ds(0, _N)], dt_v)

        for i in range(16):
            for k in range(_D // 16):
                rows0[i, pl.ds(k * 16, 16)] = zv

        def _zero(i, carry):
            pltpu.sync_copy(rows0.at[pl.ds(0, 16)],
                            agg_sh.at[pl.ds(rbase + i * 16, 16)])
            return carry
        lax.fori_loop(0, _RPT // 16, _zero, 0)
        pltpu.sync_copy(rows0.at[pl.ds(0, _RPT % 16)],
                        agg_sh.at[pl.ds(rbase + 16 * (_RPT // 16),
                                        _RPT % 16)])
        plsc.subcore_barrier()

        def _phase1(j, cnt):
            gbase = ebase + j * _GE
            pltpu.sync_copy(pk_h.at[pl.ds(gbase, _GE)], pk_g)
            pltpu.sync_copy(gd_h.at[pl.ds(gbase, _GE)], gd_g)
            for k in range(_GE // 16):
                off = k * 16
                pv = pk_g[pl.ds(off, 16)]
                sv = pv & _PMASK
                dv = lax.shift_right_logical(pv, _PACK)
                sval = plsc.load_gather(ds_v, [sv])
                tval = plsc.load_gather(dt_v, [dv])
                a = sval + tval + gd_g[pl.ds(off, 16)]
                keep = a >= 0.0
                mask_g[pl.ds(off, 16)] = jnp.where(keep, 1.0, 0.0)
                plsc.store_compressed(comp_v.at[pl.ds(cnt, 16)], pv + boff,
                                      mask=keep)
                cnt = cnt + jnp.sum(keep.astype(jnp.int32))
            pltpu.sync_copy(mask_g, m_h.at[pl.ds(gbase, _GE)])
            return cnt
        cnt = lax.fori_loop(0, _NOUT, _phase1, jnp.int32(0))

        padv = (jnp.full((16,), _TRASH, jnp.int32) + wid) << _PACK
        padv = padv + boff
        for k in range(2 * _CH // 16):
            comp_v[pl.ds(cnt + k * 16, 16)] = padv
        nit = (cnt + 2 * _CH - 1) // (2 * _CH)

        def _phase2(i, carry):
            @pl.when(i > 0)
            def _():
                pltpu.make_async_copy(x_h.at[pl.ds(0, _CH)], rows0,
                                      scat0).wait()
                pltpu.make_async_copy(x_h.at[pl.ds(0, _CH)], rows1,
                                      scat1).wait()
            c0 = i * 2 * _CH
            for k in range(_CH // 16):
                pv = comp_v[pl.ds(c0 + k * 16, 16)]
                srcs0[pl.ds(k * 16, 16)] = pv & _PMASK
                dste0[pl.ds(k * 16, 16)] = lax.shift_right_logical(pv, _PACK)
            d0 = pltpu.async_copy(x_h.at[srcs0], rows0, sem0)
            for k in range(_CH // 16):
                pv = comp_v[pl.ds(c0 + _CH + k * 16, 16)]
                srcs1[pl.ds(k * 16, 16)] = pv & _PMASK
                dste1[pl.ds(k * 16, 16)] = lax.shift_right_logical(pv, _PACK)
            d1 = pltpu.async_copy(x_h.at[srcs1], rows1, sem1)
            d0.wait()
            pltpu.async_copy(rows0, agg_sh.at[dste0], scat0, add=True)
            d1.wait()
            pltpu.async_copy(rows1, agg_sh.at[dste1], scat1, add=True)
            return carry
        lax.fori_loop(0, nit, _phase2, 0)

        @pl.when(nit > 0)
        def _():
            pltpu.make_async_copy(x_h.at[pl.ds(0, _CH)], rows0, scat0).wait()
            pltpu.make_async_copy(x_h.at[pl.ds(0, _CH)], rows1, scat1).wait()
        plsc.subcore_barrier()

        pltpu.sync_copy(agg_sh.at[pl.ds(rbase, _RPT)],
                        a_h.at[c, pl.ds(rbase, _RPT)])


def _sc_mask_agg(pk, gd, ds, dt, x, boff):
    mesh = plsc.VectorSubcoreMesh(core_axis_name="c", subcore_axis_name="s")
    fn = pl.kernel(
        functools.partial(_sc_body, boff),
        out_type=[
            jax.ShapeDtypeStruct((_E,), jnp.float32),
            jax.ShapeDtypeStruct((_NC, _NPA, _D), jnp.float32),
        ],
        mesh=mesh,
        scratch_types=[
            pltpu.VMEM((_GE,), jnp.int32),
            pltpu.VMEM((_GE,), jnp.float32),
            pltpu.VMEM((_GE,), jnp.float32),
            pltpu.VMEM((_N,), jnp.float32),
            pltpu.VMEM((_N,), jnp.float32),
            pltpu.VMEM((_CLEN,), jnp.int32),
            pltpu.VMEM((_CH,), jnp.int32),
            pltpu.VMEM((_CH,), jnp.int32),
            pltpu.VMEM((_CH,), jnp.int32),
            pltpu.VMEM((_CH,), jnp.int32),
            pltpu.VMEM((_CH, _D), jnp.float32),
            pltpu.VMEM((_CH, _D), jnp.float32),
            pltpu.VMEM_SHARED((_NPA, _D), jnp.float32),
            pltpu.SemaphoreType.DMA,
            pltpu.SemaphoreType.DMA,
            pltpu.SemaphoreType.DMA,
            pltpu.SemaphoreType.DMA,
        ],
        compiler_params=pltpu.CompilerParams(needs_layout_passes=False),
    )
    return fn(pk, gd, ds, dt, x)


def _decode_body(x_ref, agg_ref, t_ref, wd_ref, wo_ref, o_ref, l_ref):
    i = pl.program_id(0)
    a = x_ref[...] + agg_ref[0] + agg_ref[1]
    h = jnp.maximum(a @ wd_ref[...], 0.0)
    o = h @ wo_ref[...]
    o_ref[...] = o
    part = jnp.sum((o - t_ref[...]) ** 2)
    prev = jnp.where(i == 0, jnp.zeros((1, 1), jnp.float32), l_ref[...])
    l_ref[...] = prev + part


def _decode(x, agg, t, Wd, Wo, b):
    blk = 1000
    nblk = _N // blk
    return pl.pallas_call(
        _decode_body,
        grid=(nblk,),
        in_specs=[
            pl.BlockSpec((blk, _D), lambda i: (b * nblk + i, 0)),
            pl.BlockSpec((_NC, blk, _D), lambda i: (0, i, 0)),
            pl.BlockSpec((blk, _D), lambda i: (b * nblk + i, 0)),
            pl.BlockSpec((_D, _D), lambda i: (0, 0)),
            pl.BlockSpec((_D, _D), lambda i: (0, 0)),
        ],
        out_specs=[
            pl.BlockSpec((blk, _D), lambda i: (i, 0)),
            pl.BlockSpec((1, 1), lambda i: (0, 0)),
        ],
        out_shape=[
            jax.ShapeDtypeStruct((_N, _D), jnp.float32),
            jax.ShapeDtypeStruct((1, 1), jnp.float32),
        ],
    )(x, agg, t, Wd, Wo)


def kernel(inputs, targets, entire_inputs, edge_index, W1, W2, Wg, bg, Wd, Wo):
    src = edge_index[0].astype(jnp.int32)
    dst = edge_index[1].astype(jnp.int32)
    pk = src | (dst << _PACK)

    bias = bg[0] - bg[1]
    gds = []
    for b in range(_B):
        kb = jax.random.fold_in(jax.random.key(42), b)
        u = jax.random.uniform(kb, (_E, 2), minval=1e-6, maxval=1.0 - 1e-6)
        g = -jnp.log(-jnp.log(u))
        gds.append((g[:, 0] - g[:, 1]) + bias)

    us8 = jnp.zeros((8, _H), jnp.bfloat16)
    us8 = (us8.at[0].set(Wg[:_H, 0].astype(jnp.bfloat16))
               .at[1].set(Wg[:_H, 1].astype(jnp.bfloat16))
               .at[2].set(Wg[_H:, 0].astype(jnp.bfloat16))
               .at[3].set(Wg[_H:, 1].astype(jnp.bfloat16)))

    ei_p = jnp.pad(entire_inputs, ((0, 0), (0, _NPT - _N), (0, 0)))
    ddt = _node_scores(ei_p, W1, W2, us8)

    m0, a0 = _sc_mask_agg(pk, gds[0], ddt[0, 0], ddt[0, 1], inputs, 0)
    m1, a1 = _sc_mask_agg(pk, gds[1], ddt[1, 0], ddt[1, 1], inputs, _N)
    o0, l0 = _decode(inputs, a0, targets, Wd, Wo, 0)
    o1, l1 = _decode(inputs, a1, targets, Wd, Wo, 1)

    edge_masks = jnp.stack([m0, m1], axis=0)
    outputs = jnp.concatenate([o0, o1], axis=0)
    loss = (l0[0, 0] + l1[0, 0]) / (_B * _N * _D)
    return (edge_masks, outputs, loss)


# speedup vs baseline: 1.0399x; 1.0399x over previous
import jax
import jax.numpy as jnp
from jax import lax
from jax.experimental import pallas as pl
from jax.experimental.pallas import tpu as pltpu
from jax.experimental.pallas import tpu_sc as plsc

_N = 10000
_E = 320000
_B = 2
_T = 64
_D = 128
_H = 64

_NPT = 10240
_NPA = 10112
_TRASH = _N
_NC = 2
_NS = 16
_NW = _NC * _NS
_EPW = _E // _NW
_GE = 2000
_NOUT = _EPW // _GE
_CH = 48
_RPT = _NPA // _NS
_PACK = 15
_PMASK = (1 << _PACK) - 1
_CLEN = _EPW + 2 * _CH + 16


def _node_scores_body(ei_ref, w1_ref, w2_ref, us8_ref, out_ref):
    z1 = jnp.maximum(ei_ref[0] @ w1_ref[...], 0.0)
    z2 = z1 @ w2_ref[...]
    zb = z2.astype(jnp.bfloat16)
    p = lax.dot_general(us8_ref[...], zb, (((1,), (1,)), ((), ())),
                        preferred_element_type=jnp.float32)
    out_ref[0, pl.ds(0, 1)] = p[0:1] - p[1:2]
    out_ref[0, pl.ds(1, 1)] = p[2:3] - p[3:4]


def _node_scores(ei_p, W1, W2, us8):
    blk = 1280
    grid = (_B, _NPT // blk)
    return pl.pallas_call(
        _node_scores_body,
        grid=grid,
        in_specs=[
            pl.BlockSpec((1, blk, _T), lambda b, i: (b, i, 0)),
            pl.BlockSpec((_H, _H), lambda b, i: (0, 0)),
            pl.BlockSpec((_H, _H), lambda b, i: (0, 0)),
            pl.BlockSpec((8, _H), lambda b, i: (0, 0)),
        ],
        out_specs=pl.BlockSpec((1, 8, blk), lambda b, i: (b, 0, i)),
        out_shape=jax.ShapeDtypeStruct((_B, 8, _NPT), jnp.float32),
    )(ei_p, W1, W2, us8)


def _sc_body(pk_h, gd0_h, gd1_h, ds0_h, dt0_h, ds1_h, dt1_h, x_h,
             m_h, a_h,
             pk_g, gd_g, mask_g, ds_v, dt_v, comp_v, srcs0, srcs1,
             dste0, dste1, rows0, rows1, agg_sh, sem0, sem1, scat0, scat1):
    c = lax.axis_index("c")
    s = lax.axis_index("s")
    wid = c * _NS + s
    ebase = wid * _EPW
    rbase = s * _RPT

    zv = jnp.zeros((16,), jnp.float32)

    for b, (gd_h, ds_h, dt_h) in enumerate(
            ((gd0_h, ds0_h, dt0_h), (gd1_h, ds1_h, dt1_h))):
        boff = b * _N
        pltpu.sync_copy(ds_h.at[pl.ds(0, _N)], ds_v)
        pltpu.sync_copy(dt_h.at[pl.ds(0, _N)], dt_v)

        for i in range(16):
            for k in range(_D // 16):
                rows0[i, pl.ds(k * 16, 16)] = zv

        def _zero(i, carry):
            pltpu.sync_copy(rows0.at[pl.ds(0, 16)],
                            agg_sh.at[pl.ds(rbase + i * 16, 16)])
            return carry
        lax.fori_loop(0, _RPT // 16, _zero, 0)
        pltpu.sync_copy(rows0.at[pl.ds(0, _RPT % 16)],
                        agg_sh.at[pl.ds(rbase + 16 * (_RPT // 16),
                                        _RPT % 16)])
        plsc.subcore_barrier()

        def _phase1(j, cnt):
            gbase = ebase + j * _GE
            pltpu.sync_copy(pk_h.at[pl.ds(gbase, _GE)], pk_g)
            pltpu.sync_copy(gd_h.at[pl.ds(gbase, _GE)], gd_g)
            for k in range(_GE // 16):
                off = k * 16
                pv = pk_g[pl.ds(off, 16)]
                sv = pv & _PMASK
                dv = lax.shift_right_logical(pv, _PACK)
                sval = plsc.load_gather(ds_v, [sv])
                tval = plsc.load_gather(dt_v, [dv])
                a = sval + tval + gd_g[pl.ds(off, 16)]
                keep = a >= 0.0
                mask_g[pl.ds(off, 16)] = jnp.where(keep, 1.0, 0.0)
                plsc.store_compressed(comp_v.at[pl.ds(cnt, 16)], pv + boff,
                                      mask=keep)
                cnt = cnt + jnp.sum(keep.astype(jnp.int32))
            pltpu.sync_copy(mask_g, m_h.at[pl.ds(b * _E + gbase, _GE)])
            return cnt
        cnt = lax.fori_loop(0, _NOUT, _phase1, jnp.int32(0))

        padv = (jnp.full((16,), _TRASH, jnp.int32) + wid) << _PACK
        padv = padv + boff
        for k in range(2 * _CH // 16):
            comp_v[pl.ds(cnt + k * 16, 16)] = padv
        nit = (cnt + 2 * _CH - 1) // (2 * _CH)

        def _phase2(i, carry):
            @pl.when(i > 0)
            def _():
                pltpu.make_async_copy(x_h.at[pl.ds(0, _CH)], rows0,
                                      scat0).wait()
                pltpu.make_async_copy(x_h.at[pl.ds(0, _CH)], rows1,
                                      scat1).wait()
            c0 = i * 2 * _CH
            for k in range(_CH // 16):
                pv = comp_v[pl.ds(c0 + k * 16, 16)]
                srcs0[pl.ds(k * 16, 16)] = pv & _PMASK
                dste0[pl.ds(k * 16, 16)] = lax.shift_right_logical(pv, _PACK)
            d0 = pltpu.async_copy(x_h.at[srcs0], rows0, sem0)
            for k in range(_CH // 16):
                pv = comp_v[pl.ds(c0 + _CH + k * 16, 16)]
                srcs1[pl.ds(k * 16, 16)] = pv & _PMASK
                dste1[pl.ds(k * 16, 16)] = lax.shift_right_logical(pv, _PACK)
            d1 = pltpu.async_copy(x_h.at[srcs1], rows1, sem1)
            d0.wait()
            pltpu.async_copy(rows0, agg_sh.at[dste0], scat0, add=True)
            d1.wait()
            pltpu.async_copy(rows1, agg_sh.at[dste1], scat1, add=True)
            return carry
        lax.fori_loop(0, nit, _phase2, 0)

        @pl.when(nit > 0)
        def _():
            pltpu.make_async_copy(x_h.at[pl.ds(0, _CH)], rows0, scat0).wait()
            pltpu.make_async_copy(x_h.at[pl.ds(0, _CH)], rows1, scat1).wait()
        plsc.subcore_barrier()

        pltpu.sync_copy(agg_sh.at[pl.ds(rbase, _RPT)],
                        a_h.at[b * _NC + c, pl.ds(rbase, _RPT)])


def _sc_mask_agg(pk, gd0, gd1, ds0, dt0, ds1, dt1, x):
    mesh = plsc.VectorSubcoreMesh(core_axis_name="c", subcore_axis_name="s")
    fn = pl.kernel(
        _sc_body,
        out_type=[
            jax.ShapeDtypeStruct((_B * _E,), jnp.float32),
            jax.ShapeDtypeStruct((_B * _NC, _NPA, _D), jnp.float32),
        ],
        mesh=mesh,
        scratch_types=[
            pltpu.VMEM((_GE,), jnp.int32),
            pltpu.VMEM((_GE,), jnp.float32),
            pltpu.VMEM((_GE,), jnp.float32),
            pltpu.VMEM((_N,), jnp.float32),
            pltpu.VMEM((_N,), jnp.float32),
            pltpu.VMEM((_CLEN,), jnp.int32),
            pltpu.VMEM((_CH,), jnp.int32),
            pltpu.VMEM((_CH,), jnp.int32),
            pltpu.VMEM((_CH,), jnp.int32),
            pltpu.VMEM((_CH,), jnp.int32),
            pltpu.VMEM((_CH, _D), jnp.float32),
            pltpu.VMEM((_CH, _D), jnp.float32),
            pltpu.VMEM_SHARED((_NPA, _D), jnp.float32),
            pltpu.SemaphoreType.DMA,
            pltpu.SemaphoreType.DMA,
            pltpu.SemaphoreType.DMA,
            pltpu.SemaphoreType.DMA,
        ],
        compiler_params=pltpu.CompilerParams(needs_layout_passes=False),
    )
    return fn(pk, gd0, gd1, ds0, dt0, ds1, dt1, x)


def _decode_body(x_ref, agg_ref, t_ref, wd_ref, wo_ref, o_ref, l_ref):
    b = pl.program_id(0)
    i = pl.program_id(1)
    a = x_ref[...] + agg_ref[0, 0] + agg_ref[0, 1]
    h = jnp.maximum(a @ wd_ref[...], 0.0)
    o = h @ wo_ref[...]
    o_ref[...] = o
    part = jnp.sum((o - t_ref[...]) ** 2)
    first = jnp.logical_and(b == 0, i == 0)
    last = jnp.logical_and(b == _B - 1, i == pl.num_programs(1) - 1)
    prev = jnp.where(first, jnp.zeros((1, 1), jnp.float32), l_ref[...])
    tot = prev + part
    l_ref[...] = jnp.where(last, tot / (_B * _N * _D), tot)


def _decode(x, agg, t, Wd, Wo):
    blk = 1000
    nblk = _N // blk
    grid = (_B, nblk)
    return pl.pallas_call(
        _decode_body,
        grid=grid,
        in_specs=[
            pl.BlockSpec((blk, _D), lambda b, i: (b * nblk + i, 0)),
            pl.BlockSpec((1, _NC, blk, _D), lambda b, i: (b, 0, i, 0)),
            pl.BlockSpec((blk, _D), lambda b, i: (b * nblk + i, 0)),
            pl.BlockSpec((_D, _D), lambda b, i: (0, 0)),
            pl.BlockSpec((_D, _D), lambda b, i: (0, 0)),
        ],
        out_specs=[
            pl.BlockSpec((blk, _D), lambda b, i: (b * nblk + i, 0)),
            pl.BlockSpec((1, 1), lambda b, i: (0, 0)),
        ],
        out_shape=[
            jax.ShapeDtypeStruct((_B * _N, _D), jnp.float32),
            jax.ShapeDtypeStruct((1, 1), jnp.float32),
        ],
    )(x, agg, t, Wd, Wo)


def kernel(inputs, targets, entire_inputs, edge_index, W1, W2, Wg, bg, Wd, Wo):
    src = edge_index[0].astype(jnp.int32)
    dst = edge_index[1].astype(jnp.int32)
    pk = src | (dst << _PACK)

    bias = bg[0] - bg[1]
    gds = []
    for b in range(_B):
        kb = jax.random.fold_in(jax.random.key(42), b)
        u = jax.random.uniform(kb, (_E, 2), minval=1e-6, maxval=1.0 - 1e-6)
        g = -jnp.log(-jnp.log(u))
        gds.append((g[:, 0] - g[:, 1]) + bias)

    us8 = jnp.zeros((8, _H), jnp.bfloat16)
    us8 = (us8.at[0].set(Wg[:_H, 0].astype(jnp.bfloat16))
               .at[1].set(Wg[:_H, 1].astype(jnp.bfloat16))
               .at[2].set(Wg[_H:, 0].astype(jnp.bfloat16))
               .at[3].set(Wg[_H:, 1].astype(jnp.bfloat16)))

    ei_p = jnp.pad(entire_inputs, ((0, 0), (0, _NPT - _N), (0, 0)))
    ddt = _node_scores(ei_p, W1, W2, us8)

    masks_f, agg_f = _sc_mask_agg(
        pk, gds[0], gds[1],
        ddt[0, 0], ddt[0, 1], ddt[1, 0], ddt[1, 1],
        inputs)
    edge_masks = masks_f.reshape(_B, _E)
    agg = agg_f.reshape(_B, _NC, _NPA, _D)

    outputs, loss = _decode(inputs, agg, targets, Wd, Wo)
    return (edge_masks, outputs, loss[0, 0])
